# baseline (speedup 1.0000x reference)
import functools

import jax
import jax.numpy as jnp
from jax.experimental import pallas as pl
from jax.experimental.pallas import tpu as pltpu

D = 128
B = 16384
BR = 16384


def _body(idx_ref, emb_ref, gf_ref, out_ref):
    idx = idx_ref[0]
    emb_row = emb_ref[pl.ds(idx, 1), :]
    out_ref[...] = gf_ref[...] + emb_row


@jax.jit
def _view_embed(global_feat, embeddings, idx):
    grid = B // BR
    return pl.pallas_call(
        _body,
        grid_spec=pltpu.PrefetchScalarGridSpec(
            num_scalar_prefetch=1,
            grid=(grid,),
            in_specs=[
                pl.BlockSpec((3, D), lambda i, idx: (0, 0)),
                pl.BlockSpec((BR, D), lambda i, idx: (i, 0)),
            ],
            out_specs=pl.BlockSpec((BR, D), lambda i, idx: (i, 0)),
        ),
        out_shape=jax.ShapeDtypeStruct((B, D), jnp.float32),
        compiler_params=pltpu.CompilerParams(
            dimension_semantics=("arbitrary",)),
    )(idx, embeddings, global_feat)


def kernel(global_feat, embeddings, view_idx):
    idx = jnp.asarray(view_idx, dtype=jnp.int32).reshape((1,))
    return _view_embed(global_feat, embeddings, idx)


# speedup vs baseline: 1.1824x; 1.1824x over previous
import functools

import jax
import jax.numpy as jnp
from jax.experimental import pallas as pl
from jax.experimental.pallas import tpu as pltpu

D = 128
B = 16384
BR = 8192


def _body(idx_ref, emb_ref, gf_ref, out_ref):
    idx = idx_ref[0]
    emb_row = emb_ref[pl.ds(idx, 1), :]
    out_ref[...] = gf_ref[...] + emb_row


@jax.jit
def _view_embed(global_feat, embeddings, idx):
    grid = B // BR
    return pl.pallas_call(
        _body,
        grid_spec=pltpu.PrefetchScalarGridSpec(
            num_scalar_prefetch=1,
            grid=(grid,),
            in_specs=[
                pl.BlockSpec((3, D), lambda i, idx: (0, 0)),
                pl.BlockSpec((BR, D), lambda i, idx: (i, 0)),
            ],
            out_specs=pl.BlockSpec((BR, D), lambda i, idx: (i, 0)),
        ),
        out_shape=jax.ShapeDtypeStruct((B, D), jnp.float32),
        compiler_params=pltpu.CompilerParams(
            dimension_semantics=("parallel",)),
    )(idx, embeddings, global_feat)


def kernel(global_feat, embeddings, view_idx):
    idx = jnp.asarray(view_idx, dtype=jnp.int32).reshape((1,))
    return _view_embed(global_feat, embeddings, idx)


# speedup vs baseline: 1.1955x; 1.0111x over previous
import jax
import jax.numpy as jnp
from jax.experimental import pallas as pl
from jax.experimental.pallas import tpu as pltpu

D = 128
B = 16384
BR = 8192


def _body(idx_ref, emb_ref, gf_ref, out_ref):
    idx = idx_ref[0]
    emb_row = emb_ref[pl.ds(idx, 1), :]
    out_ref[...] = gf_ref[...] + emb_row


@jax.jit
def _view_embed(global_feat, embeddings, idx):
    grid = B // BR
    return pl.pallas_call(
        _body,
        grid_spec=pltpu.PrefetchScalarGridSpec(
            num_scalar_prefetch=1,
            grid=(grid,),
            in_specs=[
                pl.BlockSpec((3, D), lambda i, idx: (0, 0)),
                pl.BlockSpec((BR, D), lambda i, idx: (i, 0)),
            ],
            out_specs=pl.BlockSpec((BR, D), lambda i, idx: (i, 0)),
        ),
        out_shape=jax.ShapeDtypeStruct((B, D), jnp.float32),
        compiler_params=pltpu.CompilerParams(
            dimension_semantics=("arbitrary",)),
    )(idx, embeddings, global_feat)


def kernel(global_feat, embeddings, view_idx):
    idx = jnp.asarray(view_idx, dtype=jnp.int32).reshape((1,))
    return _view_embed(global_feat, embeddings, idx)
